# baseline (speedup 1.0000x reference)
import functools

import jax
import jax.numpy as jnp
from jax import lax
from jax.experimental import pallas as pl
from jax.experimental.pallas import tpu as pltpu
from jax.experimental.pallas import tpu_sc as plsc

GRID = 64
G3 = GRID ** 3
IMG_W, IMG_H = 640, 512
WH, HH = IMG_W // 2, IMG_H // 2
HW = WH * HH
B, C, J = 2, 8, 23
JP = 32
ROWS = B * G3
NBLK = ROWS // 128

NCORE, NSUBC = 2, 16
NW = NCORE * NSUBC
RPW = ROWS // NW
CHUNK = 4096
NCHUNK = RPW // CHUNK
NSUBCH = CHUNK // 128


def _idx_body(center_ref, cam_ref, out_ref):
    cam = pl.program_id(0)
    b = pl.program_id(1)
    n = pl.program_id(2)
    r = lax.broadcasted_iota(jnp.int32, (1, NSUBCH, 128), 1)
    q = lax.broadcasted_iota(jnp.int32, (1, NSUBCH, 128), 2)
    g = n * CHUNK + r * 128 + q
    x = g >> 12
    y = (g >> 6) & (GRID - 1)
    z = g & (GRID - 1)
    fx = (x - GRID // 2).astype(jnp.float32) * 2.0 + center_ref[b, 0]
    fy = (y - GRID // 2).astype(jnp.float32) * 2.0 + center_ref[b, 1]
    fz = (z - GRID // 2).astype(jnp.float32) * 2.0 + center_ref[b, 2]
    def _bf(t):
        return t.astype(jnp.bfloat16).astype(jnp.float32)

    fx, fy, fz = _bf(fx), _bf(fy), _bf(fz)
    m = [[_bf(cam_ref[b, cam, k, col]) for col in range(3)] for k in range(4)]
    xp = fx * m[0][0] + fy * m[1][0] + fz * m[2][0] + m[3][0]
    yp = fx * m[0][1] + fy * m[1][1] + fz * m[2][1] + m[3][1]
    zp = fx * m[0][2] + fy * m[1][2] + fz * m[2][2] + m[3][2]
    u = jnp.clip(xp / zp, 0.0, float(IMG_W - 1))
    v = jnp.clip(yp / zp, 0.0, float(IMG_H - 1))
    pix = (v * 0.5).astype(jnp.int32) * WH + (u * 0.5).astype(jnp.int32)
    out_ref[...] = pix + (b * C + cam) * HW


def _compute_idx(center, cameraMatrices):
    return pl.pallas_call(
        _idx_body,
        grid=(C, B, G3 // CHUNK),
        in_specs=[
            pl.BlockSpec(memory_space=pltpu.SMEM),
            pl.BlockSpec(memory_space=pltpu.SMEM),
        ],
        out_specs=pl.BlockSpec((1, NSUBCH, 128),
                               lambda cam, b, n: (cam, b * (G3 // CHUNK) + n, 0)),
        out_shape=jax.ShapeDtypeStruct((C, NBLK, 128), jnp.int32),
    )(center, cameraMatrices)


SPC = 16
NSUB = RPW // 128
NCH = NSUB // SPC


@functools.lru_cache(maxsize=1)
def _make_sc_gather():
    mesh = plsc.VectorSubcoreMesh(core_axis_name="c", subcore_axis_name="s")
    return functools.partial(
        pl.kernel,
        mesh=mesh,
        compiler_params=pltpu.CompilerParams(use_tc_tiling_on_sc=False),
        out_type=jax.ShapeDtypeStruct((ROWS, JP), jnp.float32),
        scratch_types=[
            pltpu.VMEM((2, C, SPC, 128), jnp.int32),
            pltpu.VMEM((2, C, 128, JP), jnp.float32),
            pltpu.SemaphoreType.DMA,
            pltpu.SemaphoreType.DMA,
        ],
    )(_sc_gather_body)


def _sc_gather_body(table_hbm, idx_hbm, out_hbm, idx_v, bufs, sem0, sem1):
    wid = lax.axis_index("s") * NCORE + lax.axis_index("c")
    wblk = wid * NSUB

    def stage(chunk):
        slab = lax.rem(chunk, 2)
        bb = pl.multiple_of(wblk + chunk * SPC, 8)
        for cam in range(C):
            pltpu.sync_copy(idx_hbm.at[cam, pl.ds(bb, SPC)],
                            idx_v.at[slab, cam])

    def fire(j, slot, sem):
        slab = lax.rem(j // SPC, 2)
        row = lax.rem(j, SPC)
        for cam in range(C):
            pltpu.async_copy(table_hbm.at[idx_v.at[slab, cam, row]],
                             bufs.at[slot, cam], sem)

    def drain(slot, sem):
        for cam in range(C):
            pltpu.make_async_copy(table_hbm.at[pl.ds(0, 128)],
                                  bufs.at[slot, cam], sem).wait()

    def acc_and_write(j, slot):
        def acc_body(r4, carry):
            for dr in range(4):
                r = r4 * 4 + dr
                for half in range(JP // 16):
                    sl = pl.ds(half * 16, 16)
                    s = bufs[slot, 0, r, sl]
                    for cam in range(1, C):
                        s = s + bufs[slot, cam, r, sl]
                    bufs[slot, 0, r, sl] = s * (1.0 / C)
            return carry

        lax.fori_loop(0, 32, acc_body, 0)
        start = pl.multiple_of((wblk + j) * 128, 128)
        pltpu.sync_copy(bufs.at[slot, 0], out_hbm.at[pl.ds(start, 128)])

    stage(0)
    fire(0, 0, sem0)

    def body(t, carry):
        j0 = 2 * t
        fire(j0 + 1, 1, sem1)
        drain(0, sem0)
        acc_and_write(j0, 0)

        @pl.when(jnp.logical_and(t != (NSUB // 2 - 1), lax.rem(t, SPC // 2) == SPC // 2 - 1))
        def _():
            stage((j0 + 2) // SPC)

        @pl.when(t != (NSUB // 2 - 1))
        def _():
            fire(j0 + 2, 0, sem0)

        drain(1, sem1)
        acc_and_write(j0 + 1, 1)
        return carry

    lax.fori_loop(0, NSUB // 2, body, 0)


def kernel(heatmaps, center, cameraMatrices):
    t = jnp.transpose(heatmaps, (0, 1, 3, 4, 2))
    t = jnp.pad(t, ((0, 0), (0, 0), (0, 0), (0, 0), (0, JP - J)))
    table = t.reshape(B * C * HW, JP)
    idx = _compute_idx(center, cameraMatrices)
    out32 = _make_sc_gather()(table, idx)
    out = jnp.moveaxis(out32.reshape(B, G3, JP), 2, 1)[:, :J, :]
    return out.reshape(B, J, GRID, GRID, GRID)


# speedup vs baseline: 1.0048x; 1.0048x over previous
import functools

import jax
import jax.numpy as jnp
from jax import lax
from jax.experimental import pallas as pl
from jax.experimental.pallas import tpu as pltpu
from jax.experimental.pallas import tpu_sc as plsc

GRID = 64
G3 = GRID ** 3
IMG_W, IMG_H = 640, 512
WH, HH = IMG_W // 2, IMG_H // 2
HW = WH * HH
B, C, J = 2, 8, 23
JP = 32
ROWS = B * G3
NBLK = ROWS // 128

NCORE, NSUBC = 2, 16
NW = NCORE * NSUBC
RPW = ROWS // NW
CHUNK = 4096
NCHUNK = RPW // CHUNK
NSUBCH = CHUNK // 128


def _idx_body(center_ref, cam_ref, out_ref):
    cam = pl.program_id(0)
    b = pl.program_id(1)
    n = pl.program_id(2)
    r = lax.broadcasted_iota(jnp.int32, (1, NSUBCH, 128), 1)
    q = lax.broadcasted_iota(jnp.int32, (1, NSUBCH, 128), 2)
    g = n * CHUNK + r * 128 + q
    x = g >> 12
    y = (g >> 6) & (GRID - 1)
    z = g & (GRID - 1)
    fx = (x - GRID // 2).astype(jnp.float32) * 2.0 + center_ref[b, 0]
    fy = (y - GRID // 2).astype(jnp.float32) * 2.0 + center_ref[b, 1]
    fz = (z - GRID // 2).astype(jnp.float32) * 2.0 + center_ref[b, 2]
    def _bf(t):
        return t.astype(jnp.bfloat16).astype(jnp.float32)

    fx, fy, fz = _bf(fx), _bf(fy), _bf(fz)
    m = [[_bf(cam_ref[b, cam, k, col]) for col in range(3)] for k in range(4)]
    xp = fx * m[0][0] + fy * m[1][0] + fz * m[2][0] + m[3][0]
    yp = fx * m[0][1] + fy * m[1][1] + fz * m[2][1] + m[3][1]
    zp = fx * m[0][2] + fy * m[1][2] + fz * m[2][2] + m[3][2]
    u = jnp.clip(xp / zp, 0.0, float(IMG_W - 1))
    v = jnp.clip(yp / zp, 0.0, float(IMG_H - 1))
    pix = (v * 0.5).astype(jnp.int32) * WH + (u * 0.5).astype(jnp.int32)
    out_ref[...] = pix + (b * C + cam) * HW


def _compute_idx(center, cameraMatrices):
    return pl.pallas_call(
        _idx_body,
        grid=(C, B, G3 // CHUNK),
        in_specs=[
            pl.BlockSpec(memory_space=pltpu.SMEM),
            pl.BlockSpec(memory_space=pltpu.SMEM),
        ],
        out_specs=pl.BlockSpec((1, NSUBCH, 128),
                               lambda cam, b, n: (cam, b * (G3 // CHUNK) + n, 0)),
        out_shape=jax.ShapeDtypeStruct((C, NBLK, 128), jnp.int32),
    )(center, cameraMatrices)


S = 64
R = 4
SPC = 32
NSUB = RPW // S


@functools.lru_cache(maxsize=1)
def _make_sc_gather():
    mesh = plsc.VectorSubcoreMesh(core_axis_name="c", subcore_axis_name="s")
    return functools.partial(
        pl.kernel,
        mesh=mesh,
        compiler_params=pltpu.CompilerParams(use_tc_tiling_on_sc=False),
        out_type=jax.ShapeDtypeStruct((ROWS, JP), jnp.float32),
        scratch_types=[
            pltpu.VMEM((2, C, SPC * S // 128, 128), jnp.int32),
            pltpu.VMEM((R, C, S, JP), jnp.float32),
        ] + [pltpu.SemaphoreType.DMA] * R,
    )(_sc_gather_body)


def _sc_gather_body(table_hbm, idx_hbm, out_hbm, idx_v, bufs, *sems):
    wid = lax.axis_index("s") * NCORE + lax.axis_index("c")
    wblk = wid * (RPW // 128)

    def stage(chunk):
        slab = lax.rem(chunk, 2)
        bb = pl.multiple_of(wblk + chunk * (SPC * S // 128), 8)
        for cam in range(C):
            pltpu.sync_copy(idx_hbm.at[cam, pl.ds(bb, SPC * S // 128)],
                            idx_v.at[slab, cam])

    def fire(j, slot, sem):
        slab = lax.rem(j // SPC, 2)
        s_in = lax.rem(j, SPC)
        blockrow = s_in // (128 // S)
        half = lax.rem(s_in, 128 // S)
        for cam in range(C):
            pltpu.async_copy(
                table_hbm.at[idx_v.at[slab, cam, blockrow, pl.ds(half * S, S)]],
                bufs.at[slot, cam], sem)

    def drain(slot, sem):
        for cam in range(C):
            pltpu.make_async_copy(table_hbm.at[pl.ds(0, S)],
                                  bufs.at[slot, cam], sem).wait()

    def acc_and_write(j, slot):
        def acc_body(r4, carry):
            for dr in range(4):
                r = r4 * 4 + dr
                for half in range(JP // 16):
                    sl = pl.ds(half * 16, 16)
                    acc = bufs[slot, 0, r, sl]
                    for cam in range(1, C):
                        acc = acc + bufs[slot, cam, r, sl]
                    bufs[slot, 0, r, sl] = acc * (1.0 / C)
            return carry

        lax.fori_loop(0, S // 4, acc_body, 0)
        start = pl.multiple_of(wid * RPW + j * S, S)
        pltpu.sync_copy(bufs.at[slot, 0], out_hbm.at[pl.ds(start, S)])

    stage(0)
    for p in range(R - 1):
        fire(jnp.int32(p), p, sems[p])

    def body(t, carry):
        for p in range(R):
            j = t * R + p
            jf = j + R - 1

            @pl.when(jnp.logical_and(jf < NSUB, lax.rem(jf, SPC) == 0))
            def _():
                stage(jf // SPC)

            @pl.when(jf < NSUB)
            def _():
                fire(jf, (p + R - 1) % R, sems[(p + R - 1) % R])

            drain(p, sems[p])
            acc_and_write(j, p)
        return carry

    lax.fori_loop(0, NSUB // R, body, 0)


def kernel(heatmaps, center, cameraMatrices):
    t = jnp.transpose(heatmaps, (0, 1, 3, 4, 2))
    t = jnp.pad(t, ((0, 0), (0, 0), (0, 0), (0, 0), (0, JP - J)))
    table = t.reshape(B * C * HW, JP)
    idx = _compute_idx(center, cameraMatrices)
    out32 = _make_sc_gather()(table, idx)
    out = jnp.moveaxis(out32.reshape(B, G3, JP), 2, 1)[:, :J, :]
    return out.reshape(B, J, GRID, GRID, GRID)


# speedup vs baseline: 1.4500x; 1.4430x over previous
import functools

import jax
import jax.numpy as jnp
from jax import lax
from jax.experimental import pallas as pl
from jax.experimental.pallas import tpu as pltpu
from jax.experimental.pallas import tpu_sc as plsc

GRID = 64
G3 = GRID ** 3
IMG_W, IMG_H = 640, 512
WH, HH = IMG_W // 2, IMG_H // 2
HW = WH * HH
B, C, J = 2, 8, 23
JP = 32
ROWS = B * G3
NBLK = ROWS // 128

NCORE, NSUBC = 2, 16
NW = NCORE * NSUBC
RPW = ROWS // NW
CHUNK = 4096
NCHUNK = RPW // CHUNK
NSUBCH = CHUNK // 128


def _idx_body(center_ref, cam_ref, out_ref):
    cam = pl.program_id(0)
    b = pl.program_id(1)
    n = pl.program_id(2)
    r = lax.broadcasted_iota(jnp.int32, (1, NSUBCH, 128), 1)
    q = lax.broadcasted_iota(jnp.int32, (1, NSUBCH, 128), 2)
    g = n * CHUNK + r * 128 + q
    x = g >> 12
    y = (g >> 6) & (GRID - 1)
    z = g & (GRID - 1)
    fx = (x - GRID // 2).astype(jnp.float32) * 2.0 + center_ref[b, 0]
    fy = (y - GRID // 2).astype(jnp.float32) * 2.0 + center_ref[b, 1]
    fz = (z - GRID // 2).astype(jnp.float32) * 2.0 + center_ref[b, 2]
    def _bf(t):
        return t.astype(jnp.bfloat16).astype(jnp.float32)

    fx, fy, fz = _bf(fx), _bf(fy), _bf(fz)
    m = [[_bf(cam_ref[b, cam, k, col]) for col in range(3)] for k in range(4)]
    xp = fx * m[0][0] + fy * m[1][0] + fz * m[2][0] + m[3][0]
    yp = fx * m[0][1] + fy * m[1][1] + fz * m[2][1] + m[3][1]
    zp = fx * m[0][2] + fy * m[1][2] + fz * m[2][2] + m[3][2]
    u = jnp.clip(xp / zp, 0.0, float(IMG_W - 1))
    v = jnp.clip(yp / zp, 0.0, float(IMG_H - 1))
    pix = (v * 0.5).astype(jnp.int32) * WH + (u * 0.5).astype(jnp.int32)
    out_ref[...] = pix + (b * C + cam) * HW


def _compute_idx(center, cameraMatrices):
    return pl.pallas_call(
        _idx_body,
        grid=(C, B, G3 // CHUNK),
        in_specs=[
            pl.BlockSpec(memory_space=pltpu.SMEM),
            pl.BlockSpec(memory_space=pltpu.SMEM),
        ],
        out_specs=pl.BlockSpec((1, NSUBCH, 128),
                               lambda cam, b, n: (cam, b * (G3 // CHUNK) + n, 0)),
        out_shape=jax.ShapeDtypeStruct((C, NBLK, 128), jnp.int32),
    )(center, cameraMatrices)


S = 64
R = 4
SPC = 32
NSUB = RPW // S


@functools.lru_cache(maxsize=1)
def _make_sc_gather():
    mesh = plsc.VectorSubcoreMesh(core_axis_name="c", subcore_axis_name="s")
    return functools.partial(
        pl.kernel,
        mesh=mesh,
        compiler_params=pltpu.CompilerParams(use_tc_tiling_on_sc=False),
        out_type=jax.ShapeDtypeStruct((ROWS, JP), jnp.float32),
        scratch_types=[
            pltpu.VMEM((2, C, SPC * S // 128, 128), jnp.int32),
            pltpu.VMEM((R, C, S, JP // 2), jnp.int32),
            pltpu.VMEM((S, JP), jnp.float32),
        ] + [pltpu.SemaphoreType.DMA] * R,
    )(_sc_gather_body)


def _sc_gather_body(table_hbm, idx_hbm, out_hbm, idx_v, bufs, obuf, *sems):
    wid = lax.axis_index("s") * NCORE + lax.axis_index("c")
    wblk = wid * (RPW // 128)

    def stage(chunk):
        slab = lax.rem(chunk, 2)
        bb = pl.multiple_of(wblk + chunk * (SPC * S // 128), 8)
        for cam in range(C):
            pltpu.sync_copy(idx_hbm.at[cam, pl.ds(bb, SPC * S // 128)],
                            idx_v.at[slab, cam])

    def fire(j, slot, sem):
        slab = lax.rem(j // SPC, 2)
        s_in = lax.rem(j, SPC)
        blockrow = s_in // (128 // S)
        half = lax.rem(s_in, 128 // S)
        for cam in range(C):
            pltpu.async_copy(
                table_hbm.at[idx_v.at[slab, cam, blockrow, pl.ds(half * S, S)]],
                bufs.at[slot, cam], sem)

    def drain(slot, sem):
        for cam in range(C):
            pltpu.make_async_copy(table_hbm.at[pl.ds(0, S)],
                                  bufs.at[slot, cam], sem).wait()

    def acc_and_write(j, slot):
        himask = jnp.full((16,), -65536, jnp.int32)
        sh16 = jnp.full((16,), 16, jnp.int32)

        def acc_body(r4, carry):
            for dr in range(4):
                r = r4 * 4 + dr
                w = bufs[slot, 0, r, :]
                acc_e = lax.bitcast_convert_type(lax.shift_left(w, sh16), jnp.float32)
                acc_o = lax.bitcast_convert_type(w & himask, jnp.float32)
                for cam in range(1, C):
                    w = bufs[slot, cam, r, :]
                    acc_e = acc_e + lax.bitcast_convert_type(
                        lax.shift_left(w, sh16), jnp.float32)
                    acc_o = acc_o + lax.bitcast_convert_type(w & himask, jnp.float32)
                obuf[r, pl.ds(0, 16)] = acc_e * (1.0 / C)
                obuf[r, pl.ds(16, 16)] = acc_o * (1.0 / C)
            return carry

        lax.fori_loop(0, S // 4, acc_body, 0)
        start = pl.multiple_of(wid * RPW + j * S, S)
        pltpu.sync_copy(obuf, out_hbm.at[pl.ds(start, S)])

    stage(0)
    for p in range(R - 1):
        fire(jnp.int32(p), p, sems[p])

    def body(t, carry):
        for p in range(R):
            j = t * R + p
            jf = j + R - 1

            @pl.when(jnp.logical_and(jf < NSUB, lax.rem(jf, SPC) == 0))
            def _():
                stage(jf // SPC)

            @pl.when(jf < NSUB)
            def _():
                fire(jf, (p + R - 1) % R, sems[(p + R - 1) % R])

            drain(p, sems[p])
            acc_and_write(j, p)
        return carry

    lax.fori_loop(0, NSUB // R, body, 0)


def kernel(heatmaps, center, cameraMatrices):
    t = jnp.transpose(heatmaps.astype(jnp.bfloat16), (0, 1, 3, 4, 2))
    t = jnp.pad(t, ((0, 0), (0, 0), (0, 0), (0, 0), (0, JP - J)))
    table = lax.bitcast_convert_type(
        t.reshape(B * C * HW, JP // 2, 2), jnp.int32)
    idx = _compute_idx(center, cameraMatrices)
    out32 = _make_sc_gather()(table, idx)
    cols = jnp.array([(j // 2) + 16 * (j % 2) for j in range(J)], jnp.int32)
    out = jnp.moveaxis(out32.reshape(B, G3, JP)[:, :, cols], 2, 1)
    return out.reshape(B, J, GRID, GRID, GRID)
